# baseline (speedup 1.0000x reference)
import functools

import jax
import jax.numpy as jnp
from jax import lax
from jax.experimental import pallas as pl
from jax.experimental.pallas import tpu as pltpu
from jax.experimental.pallas import tpu_sc as plsc

BATCH = 16384
EMBED = 32
HIDDEN = 64

_info = plsc.get_sparse_core_info()
_NC, _NS = _info.num_cores, _info.num_subcores
_NW = _NC * _NS
_BPW = BATCH // _NW

_mesh = plsc.VectorSubcoreMesh(core_axis_name="c", subcore_axis_name="s")


@functools.partial(
    pl.kernel,
    mesh=_mesh,
    out_type=(
        jax.ShapeDtypeStruct((BATCH, EMBED), jnp.float32),
        jax.ShapeDtypeStruct((BATCH, EMBED), jnp.float32),
    ),
    scratch_types=[
        pltpu.VMEM((_BPW,), jnp.int32),
        pltpu.VMEM((_BPW,), jnp.int32),
        pltpu.VMEM((_BPW, EMBED), jnp.float32),
        pltpu.VMEM((_BPW, EMBED), jnp.float32),
        pltpu.SemaphoreType.DMA,
        pltpu.SemaphoreType.DMA,
    ],
    compiler_params=pltpu.CompilerParams(use_tc_tiling_on_sc=False),
)
def _sc_gather(uid_hbm, pid_hbm, ut_hbm, pt_hbm, uo_hbm, po_hbm,
               uidx_v, pidx_v, urows_v, prows_v, sem_u, sem_p):
    wid = lax.axis_index("s") * _NC + lax.axis_index("c")
    base = wid * _BPW
    pltpu.sync_copy(uid_hbm.at[pl.ds(base, _BPW)], uidx_v)
    pltpu.sync_copy(pid_hbm.at[pl.ds(base, _BPW)], pidx_v)
    cu = pltpu.async_copy(ut_hbm.at[uidx_v], urows_v, sem_u)
    cp = pltpu.async_copy(pt_hbm.at[pidx_v], prows_v, sem_p)
    cu.wait()
    cp.wait()
    pltpu.sync_copy(urows_v, uo_hbm.at[pl.ds(base, _BPW)])
    pltpu.sync_copy(prows_v, po_hbm.at[pl.ds(base, _BPW)])


_BLK = 2048


def _mlp_body(u_ref, p_ref, w1u_ref, w1p_ref, b1_ref, w2_ref, b2_ref, o_ref):
    x = (jnp.dot(u_ref[...], w1u_ref[...], preferred_element_type=jnp.float32)
         + jnp.dot(p_ref[...], w1p_ref[...], preferred_element_type=jnp.float32)
         + b1_ref[...])
    x = jnp.maximum(x, 0.0)
    o_ref[...] = jnp.sum(x * w2_ref[...], axis=1, keepdims=True) + b2_ref[...]


def _mlp(u_rows, p_rows, w1u, w1p, b1_2d, w2t, b2_2d):
    grid = (BATCH // _BLK,)
    return pl.pallas_call(
        _mlp_body,
        grid=grid,
        in_specs=[
            pl.BlockSpec((_BLK, EMBED), lambda i: (i, 0)),
            pl.BlockSpec((_BLK, EMBED), lambda i: (i, 0)),
            pl.BlockSpec((EMBED, HIDDEN), lambda i: (0, 0)),
            pl.BlockSpec((EMBED, HIDDEN), lambda i: (0, 0)),
            pl.BlockSpec((1, HIDDEN), lambda i: (0, 0)),
            pl.BlockSpec((1, HIDDEN), lambda i: (0, 0)),
            pl.BlockSpec((1, 1), lambda i: (0, 0)),
        ],
        out_specs=pl.BlockSpec((_BLK, 1), lambda i: (i, 0)),
        out_shape=jax.ShapeDtypeStruct((BATCH, 1), jnp.float32),
    )(u_rows, p_rows, w1u, w1p, b1_2d, w2t, b2_2d)


def kernel(user_ids, post_ids, user_table, post_table, W1, b1, W2, b2):
    u_rows, p_rows = _sc_gather(user_ids, post_ids, user_table, post_table)
    return _mlp(
        u_rows, p_rows,
        W1[:EMBED], W1[EMBED:],
        b1.reshape(1, HIDDEN),
        W2.reshape(1, HIDDEN),
        b2.reshape(1, 1),
    )
